# baseline (speedup 1.0000x reference)
import functools

import jax
import jax.numpy as jnp
from jax import lax
from jax.experimental import pallas as pl
from jax.experimental.pallas import tpu as pltpu
from jax.experimental.pallas import tpu_sc as plsc

N = 10000
E = 320000
D = 128
NC = 2
NS = 16
L = 16
NW = NC * NS
NP = 10240
NPT = NP // NW
CHUNK = 256
GB = 256
OCAP = CHUNK + L


def _mm_relu_body(x_ref, w_ref, b_ref, o_ref):
    o_ref[...] = jnp.maximum(
        jnp.dot(x_ref[...], w_ref[...], preferred_element_type=jnp.float32)
        + b_ref[...], 0.0)


def _mid_body(x_ref, ng_ref, ws_ref, wn_ref, b_ref, wp_ref, bp_ref,
              h1_ref, fs1_ref):
    ng = ng_ref[...]
    ng = jnp.where(ng == -jnp.inf, 0.0, ng)
    h1 = (jnp.dot(x_ref[...], ws_ref[...], preferred_element_type=jnp.float32)
          + jnp.dot(ng, wn_ref[...], preferred_element_type=jnp.float32)
          + b_ref[...])
    h1 = jnp.maximum(h1, 0.0)
    h1_ref[...] = h1
    fs1_ref[...] = jnp.maximum(
        jnp.dot(h1, wp_ref[...], preferred_element_type=jnp.float32)
        + bp_ref[...], 0.0)


def _out_body(h1_ref, ng_ref, ws_ref, wn_ref, b_ref, o_ref):
    ng = ng_ref[...]
    ng = jnp.where(ng == -jnp.inf, 0.0, ng)
    o_ref[...] = (
        jnp.dot(h1_ref[...], ws_ref[...], preferred_element_type=jnp.float32)
        + jnp.dot(ng, wn_ref[...], preferred_element_type=jnp.float32)
        + b_ref[...])


_BM = NP // 4


def _full(shape):
    return pl.BlockSpec(shape, lambda i: (0, 0))


def _rows(width):
    return pl.BlockSpec((_BM, width), lambda i: (i, 0))


def _mm_relu(x, w, b):
    dout = w.shape[1]
    return pl.pallas_call(
        _mm_relu_body,
        grid=(NP // _BM,),
        in_specs=[_rows(D), _full((D, dout)), _full((1, dout))],
        out_specs=_rows(dout),
        out_shape=jax.ShapeDtypeStruct((NP, dout), jnp.float32),
    )(x, w, b.reshape(1, -1))


def _mid_stage(x, ng, ws, wn, b, wp, bp):
    return pl.pallas_call(
        _mid_body,
        grid=(NP // _BM,),
        in_specs=[_rows(D), _rows(D), _full((D, D)), _full((D, D)),
                  _full((1, D)), _full((D, D)), _full((1, D))],
        out_specs=[_rows(D), _rows(D)],
        out_shape=[jax.ShapeDtypeStruct((NP, D), jnp.float32),
                   jax.ShapeDtypeStruct((NP, D), jnp.float32)],
    )(x, ng, ws, wn, b.reshape(1, -1), wp, bp.reshape(1, -1))


def _out_stage(h1, ng, ws, wn, b):
    dout = ws.shape[1]
    return pl.pallas_call(
        _out_body,
        grid=(NP // _BM,),
        in_specs=[_rows(D), _rows(D), _full((D, dout)), _full((D, dout)),
                  _full((1, dout))],
        out_specs=_rows(dout),
        out_shape=jax.ShapeDtypeStruct((NP, dout), jnp.float32),
    )(h1, ng, ws, wn, b.reshape(1, -1))


@functools.partial(
    pl.kernel,
    out_type=jax.ShapeDtypeStruct((NP, D), jnp.float32),
    mesh=plsc.VectorSubcoreMesh(core_axis_name="c", subcore_axis_name="s"),
    compiler_params=pltpu.CompilerParams(needs_layout_passes=False),
    scratch_types=[
        pltpu.VMEM((NPT + 1, D), jnp.float32),
        pltpu.VMEM((CHUNK,), jnp.int32),
        pltpu.VMEM((CHUNK,), jnp.int32),
        pltpu.VMEM((GB,), jnp.int32),
        pltpu.VMEM((GB,), jnp.int32),
        pltpu.VMEM((OCAP,), jnp.int32),
        pltpu.VMEM((OCAP,), jnp.int32),
        pltpu.VMEM((GB, D), jnp.float32),
        pltpu.SemaphoreType.DMA,
    ],
)
def _segmax(feat, src, dst, out, acc, csrc, cdst, pas, pad, pbs, pbd, rows,
            sem):
    wid = lax.axis_index("s") * NC + lax.axis_index("c")
    lo = wid * NPT

    neg = jnp.full((L,), -jnp.inf, jnp.float32)

    def init_row(r, carry):
        for c in range(D // L):
            acc[r, pl.ds(c * L, L)] = neg
        return carry

    lax.fori_loop(0, NPT + 1, init_row, 0)

    zero = jnp.zeros((L,), jnp.int32)
    trash = jnp.full((L,), NPT, jnp.int32)
    for i in range(GB // L):
        pas[pl.ds(i * L, L)] = zero
        pad[pl.ds(i * L, L)] = trash
    for i in range(OCAP // L):
        pbs[pl.ds(i * L, L)] = zero
        pbd[pl.ds(i * L, L)] = trash

    def accumulate(count):
        pltpu.async_copy(feat.at[pas], rows, sem).wait()

        def grp_body(g, carry):
            dlv = pad[pl.ds(g * L, L)]
            for j in range(L):
                r = dlv[j]
                for c in range(D // L):
                    s = pl.ds(c * L, L)
                    acc[r, s] = jnp.maximum(acc[r, s], rows[g * L + j, s])
            return carry

        lax.fori_loop(0, (count + L - 1) // L, grp_body, 0)

    def chunk_body(ch, np_):
        base = ch * CHUNK
        pltpu.sync_copy(src.at[pl.ds(base, CHUNK)], csrc)
        pltpu.sync_copy(dst.at[pl.ds(base, CHUNK)], cdst)
        lo_v = jnp.broadcast_to(lo, (L,))
        npt_v = jnp.full((L,), NPT, jnp.int32)
        gb_v = jnp.full((L,), GB, jnp.int32)
        zero_v = jnp.zeros((L,), jnp.int32)
        for g in range(CHUNK // L):
            sl = pl.ds(g * L, L)
            d = cdst[sl]
            s_ = csrc[sl]
            dl = d - lo_v
            m = (dl >= zero_v) & (dl < npt_v)
            pc = plsc.cumsum(m.astype(jnp.int32))
            idx = jnp.broadcast_to(np_ - 1, (L,)) + pc
            ma = m & (idx < gb_v)
            mb = m & (idx >= gb_v)
            plsc.store_scatter(pas, [idx], s_, mask=ma)
            plsc.store_scatter(pad, [idx], dl, mask=ma)
            plsc.store_scatter(pbs, [idx - gb_v], s_, mask=mb)
            plsc.store_scatter(pbd, [idx - gb_v], dl, mask=mb)
            np_ = np_ + pc[L - 1]

        def drain(n):
            accumulate(jnp.int32(GB))
            for i in range(CHUNK // L):
                s0 = pl.ds(i * L, L)
                pas[s0] = pbs[s0]
                pad[s0] = pbd[s0]
            return n - GB

        return lax.cond(np_ >= GB, drain, lambda n: n, np_)

    np_ = lax.fori_loop(0, E // CHUNK, chunk_body, jnp.int32(0))
    lax.cond(np_ > 0,
             lambda n: (accumulate(n), jnp.int32(0))[1],
             lambda n: n, np_)

    pltpu.sync_copy(acc.at[pl.ds(0, NPT)], out.at[pl.ds(lo, NPT)])


def kernel(features, edge_index, W_pool0, b_pool0, W_self0, W_neigh0, bias0,
           W_pool1, b_pool1, W_self1, W_neigh1, bias1):
    src = edge_index[0]
    dst = edge_index[1]
    xp = jnp.pad(features, ((0, NP - N), (0, 0)))

    fs0 = _mm_relu(xp, W_pool0, b_pool0)
    ng0 = _segmax(fs0, src, dst)
    h1, fs1 = _mid_stage(xp, ng0, W_self0, W_neigh0, bias0, W_pool1, b_pool1)
    ng1 = _segmax(fs1, src, dst)
    out = _out_stage(h1, ng1, W_self1, W_neigh1, bias1)
    return out[:N]
